# baseline (speedup 1.0000x reference)
import functools

import jax
import jax.numpy as jnp
from jax import lax
from jax.experimental import pallas as pl
from jax.experimental.pallas import tpu as pltpu
from jax.experimental.pallas import tpu_sc as plsc

_B = 128
_N = 4096
_K = 8
_NBLK = 8
_BO = _N // _NBLK

_L = 16
_NCHUNK = _N // _L
_NC = 2
_NS = 16
_NW = _NC * _NS
_RPW = _B // _NW

_NEG = float("-inf")
_POS = float("inf")


def _mm_body(field_ref, w_ref, b_ref, out_ref):
    blk = lax.dot_general(
        field_ref[...], w_ref[...], (((1,), (1,)), ((), ())),
        preferred_element_type=jnp.float32,
    )
    out_ref[...] = blk + b_ref[...].reshape(1, _BO)


def _scores(field, W, b):
    return pl.pallas_call(
        _mm_body,
        grid=(_NBLK,),
        in_specs=[
            pl.BlockSpec((_B, _N), lambda i: (0, 0)),
            pl.BlockSpec((_BO, _N), lambda i: (i, 0)),
            pl.BlockSpec((_BO,), lambda i: (i,)),
        ],
        out_specs=pl.BlockSpec((_B, _BO), lambda i: (0, i)),
        out_shape=jax.ShapeDtypeStruct((_B, _N), jnp.float32),
        compiler_params=pltpu.CompilerParams(
            dimension_semantics=("arbitrary",),
        ),
    )(field, W, b)


def _sc_topk(scores_flat, field_flat):
    mesh = plsc.VectorSubcoreMesh(core_axis_name="c", subcore_axis_name="s")

    @functools.partial(
        pl.kernel,
        mesh=mesh,
        out_type=jax.ShapeDtypeStruct((_B * _K,), jnp.float32),
        compiler_params=pltpu.CompilerParams(needs_layout_passes=False),
        scratch_types=[
            pltpu.VMEM((_N,), jnp.float32),
            pltpu.VMEM((_RPW * _L,), jnp.int32),
            pltpu.VMEM((_RPW * _L,), jnp.float32),
            pltpu.SemaphoreType.DMA,
        ],
    )
    def body(scores_hbm, field_hbm, out_hbm, row_v, idx_v, val_v, sem):
        wid = lax.axis_index("s") * _NC + lax.axis_index("c")
        lane = lax.broadcasted_iota(jnp.int32, (_L,), 0)
        for r in range(_RPW):
            row = wid * _RPW + r
            pltpu.sync_copy(scores_hbm.at[pl.ds(row * _N, _N)], row_v)

            def chunk(c, carry):
                top_v, top_i, thr = carry
                v = row_v[pl.ds(c * _L, _L)]
                m = jnp.max(v)

                def merge(args):
                    tv, ti = args
                    gi = lane + c * _L
                    v_d, i_d = plsc.sort_key_val(v, gi, descending=True)
                    take = tv >= v_d
                    mv = jnp.maximum(tv, v_d)
                    mi = jnp.where(take, ti, i_d)
                    ntv, nti = plsc.sort_key_val(mv, mi)
                    nthr = jnp.min(jnp.where(lane >= _K, ntv, _POS))
                    return ntv, nti, nthr

                def skip(args):
                    tv, ti = args
                    return tv, ti, thr

                return lax.cond(m > thr, merge, skip, (top_v, top_i))

            init = (
                jnp.full((_L,), _NEG, jnp.float32),
                jnp.zeros((_L,), jnp.int32),
                jnp.float32(_NEG),
            )
            _, top_i, _ = lax.fori_loop(0, _NCHUNK, chunk, init)
            idx_v[pl.ds(r * _L, _L)] = lax.rev(top_i, (0,)) + row * _N
        pltpu.async_copy(field_hbm.at[idx_v], val_v, sem).wait()
        for r in range(_RPW):
            row = wid * _RPW + r
            pltpu.sync_copy(
                val_v.at[pl.ds(r * _L, _K)], out_hbm.at[pl.ds(row * _K, _K)]
            )

    return body(scores_flat, field_flat)


def kernel(field, W, b, training):
    del training
    scores = _scores(field, W, b)
    out = _sc_topk(scores.reshape(-1), field.reshape(-1))
    return out.reshape(_B, _K)


# speedup vs baseline: 1.3544x; 1.3544x over previous
import functools

import jax
import jax.numpy as jnp
from jax import lax
from jax.experimental import pallas as pl
from jax.experimental.pallas import tpu as pltpu
from jax.experimental.pallas import tpu_sc as plsc

_B = 128
_N = 4096
_K = 8
_NBLK = 8
_BO = _N // _NBLK

_L = 16
_NCHUNK = _N // _L
_NC = 2
_NS = 16
_NW = _NC * _NS
_RPW = _B // _NW

_NEG = float("-inf")
_POS = float("inf")


def _mm_body(field_ref, w_ref, b_ref, out_ref):
    blk = lax.dot_general(
        field_ref[...], w_ref[...], (((1,), (1,)), ((), ())),
        preferred_element_type=jnp.float32,
    )
    out_ref[...] = blk + b_ref[...].reshape(1, _BO)


def _scores(field, W, b):
    return pl.pallas_call(
        _mm_body,
        grid=(_NBLK,),
        in_specs=[
            pl.BlockSpec((_B, _N), lambda i: (0, 0)),
            pl.BlockSpec((_BO, _N), lambda i: (i, 0)),
            pl.BlockSpec((_BO,), lambda i: (i,)),
        ],
        out_specs=pl.BlockSpec((_B, _BO), lambda i: (0, i)),
        out_shape=jax.ShapeDtypeStruct((_B, _N), jnp.float32),
        compiler_params=pltpu.CompilerParams(
            dimension_semantics=("arbitrary",),
        ),
    )(field, W, b)


def _sc_topk(scores_flat, field_flat):
    mesh = plsc.VectorSubcoreMesh(core_axis_name="c", subcore_axis_name="s")

    @functools.partial(
        pl.kernel,
        mesh=mesh,
        out_type=jax.ShapeDtypeStruct((_B * _K,), jnp.float32),
        compiler_params=pltpu.CompilerParams(needs_layout_passes=False),
        scratch_types=[
            pltpu.VMEM((_N,), jnp.float32),
            pltpu.VMEM((_N,), jnp.float32),
            pltpu.VMEM((_N + _L,), jnp.int32),
            pltpu.VMEM((_RPW * _L,), jnp.int32),
            pltpu.VMEM((_RPW * _L,), jnp.float32),
            pltpu.SemaphoreType.DMA,
            pltpu.SemaphoreType.DMA,
            pltpu.SemaphoreType.DMA,
        ],
    )
    def body(
        scores_hbm, field_hbm, out_hbm, row_a, row_b, cand_i, idx_v, val_v,
        sem_a, sem_b, sem_g,
    ):
        wid = lax.axis_index("s") * _NC + lax.axis_index("c")
        lane = lax.broadcasted_iota(jnp.int32, (_L,), 0)
        bufs = (row_a, row_b)
        sems = (sem_a, sem_b)
        row0 = wid * _RPW
        pending = {
            0: pltpu.async_copy(scores_hbm.at[pl.ds(row0 * _N, _N)], row_a, sem_a)
        }
        for r in range(_RPW):
            row = row0 + r
            row_v = bufs[r % 2]
            pending.pop(r).wait()
            if r + 1 < _RPW:
                pending[r + 1] = pltpu.async_copy(
                    scores_hbm.at[pl.ds((row + 1) * _N, _N)],
                    bufs[(r + 1) % 2],
                    sems[(r + 1) % 2],
                )

            def p1(c, m):
                return jnp.maximum(m, row_v[pl.ds(c * _L, _L)])

            mx = lax.fori_loop(0, _NCHUNK, p1, jnp.full((_L,), _NEG, jnp.float32))

            ms, _ = plsc.sort_key_val(mx, lane)
            thr = ms[_K]
            thr_b = jnp.full((_L,), thr, jnp.float32)

            def p3(c, off):
                v = row_v[pl.ds(c * _L, _L)]
                msk = v >= thr_b
                plsc.store_compressed(
                    cand_i.at[pl.ds(off, _L)], lane + c * _L, mask=msk
                )
                cnt = plsc.all_reduce_population_count(msk)
                return off + cnt[0]

            ncand = lax.fori_loop(0, _NCHUNK, p3, jnp.int32(0))

            def p4(j, carry):
                tv, ti = carry
                base = j * _L
                ci = cand_i[pl.ds(base, _L)]
                valid = (lane + base) < ncand
                ci = jnp.where(valid, ci, 0)
                (cv,) = (plsc.load_gather(row_v, [ci]),)
                cv = jnp.where(valid, cv, _NEG)
                v_d, i_d = plsc.sort_key_val(cv, ci, descending=True)
                take = tv >= v_d
                mv = jnp.maximum(tv, v_d)
                mi = jnp.where(take, ti, i_d)
                return tuple(plsc.sort_key_val(mv, mi))

            nchunk = (ncand + _L - 1) // _L
            init = (
                jnp.full((_L,), _NEG, jnp.float32),
                jnp.zeros((_L,), jnp.int32),
            )
            _, top_i = lax.fori_loop(0, nchunk, p4, init)
            idx_v[pl.ds(r * _L, _L)] = lax.rev(top_i, (0,)) + row * _N
        pltpu.async_copy(field_hbm.at[idx_v], val_v, sem_g).wait()
        for r in range(_RPW):
            row = row0 + r
            pltpu.sync_copy(
                val_v.at[pl.ds(r * _L, _K)], out_hbm.at[pl.ds(row * _K, _K)]
            )

    return body(scores_flat, field_flat)


def kernel(field, W, b, training):
    del training
    scores = _scores(field, W, b)
    out = _sc_topk(scores.reshape(-1), field.reshape(-1))
    return out.reshape(_B, _K)


# speedup vs baseline: 2.2509x; 1.6619x over previous
import jax
import jax.numpy as jnp
from jax import lax
from jax.experimental import pallas as pl
from jax.experimental.pallas import tpu as pltpu

_B = 128
_N = 4096
_K = 8
_NBLK = 8
_BO = _N // _NBLK
_CW = 128
_NC = _NBLK * _CW

_NEG = float("-inf")


def _body(field_ref, fblk_ref, w_ref, b_ref, out_ref, cs_ref, cf_ref):
    i = pl.program_id(0)
    f = field_ref[...]
    wblk = w_ref[...]
    blk = lax.dot_general(
        f, wblk, (((1,), (1,)), ((), ())), preferred_element_type=jnp.float32
    )
    s = blk + b_ref[...].reshape(1, _BO)
    fblk = fblk_ref[...]
    col = lax.broadcasted_iota(jnp.int32, (_B, _BO), 1)

    loc_s, loc_f = [], []
    for k in range(_K):
        m = jnp.max(s, axis=1, keepdims=True)
        idx = jnp.min(jnp.where(s >= m, col, _BO), axis=1, keepdims=True)
        hit = col == idx
        loc_s.append(m)
        loc_f.append(jnp.max(jnp.where(hit, fblk, _NEG), axis=1, keepdims=True))
        if k < _K - 1:
            s = jnp.where(hit, _NEG, s)
    pad_s = jnp.full((_B, _CW - _K), _NEG, jnp.float32)
    pad_f = jnp.zeros((_B, _CW - _K), jnp.float32)
    cs_ref[:, pl.ds(i * _CW, _CW)] = jnp.concatenate(loc_s + [pad_s], axis=1)
    cf_ref[:, pl.ds(i * _CW, _CW)] = jnp.concatenate(loc_f + [pad_f], axis=1)

    @pl.when(i == _NBLK - 1)
    def _merge():
        cs = cs_ref[...]
        cf = cf_ref[...]
        ccol = lax.broadcasted_iota(jnp.int32, (_B, _NC), 1)
        for k in range(_K):
            m = jnp.max(cs, axis=1, keepdims=True)
            idx = jnp.min(jnp.where(cs >= m, ccol, _NC), axis=1, keepdims=True)
            hit = ccol == idx
            out_ref[:, k] = jnp.max(jnp.where(hit, cf, _NEG), axis=1)
            if k < _K - 1:
                cs = jnp.where(hit, _NEG, cs)


def kernel(field, W, b, training):
    del training
    return pl.pallas_call(
        _body,
        grid=(_NBLK,),
        in_specs=[
            pl.BlockSpec((_B, _N), lambda i: (0, 0)),
            pl.BlockSpec((_B, _BO), lambda i: (0, i)),
            pl.BlockSpec((_BO, _N), lambda i: (i, 0)),
            pl.BlockSpec((_BO,), lambda i: (i,)),
        ],
        out_specs=pl.BlockSpec((_B, _K), lambda i: (0, 0)),
        out_shape=jax.ShapeDtypeStruct((_B, _K), jnp.float32),
        scratch_shapes=[
            pltpu.VMEM((_B, _NC), jnp.float32),
            pltpu.VMEM((_B, _NC), jnp.float32),
        ],
        compiler_params=pltpu.CompilerParams(
            dimension_semantics=("arbitrary",),
        ),
    )(field, field, W, b)


# speedup vs baseline: 2.7313x; 1.2134x over previous
import jax
import jax.numpy as jnp
from jax import lax
from jax.experimental import pallas as pl
from jax.experimental.pallas import tpu as pltpu

_B = 128
_N = 4096
_K = 8
_NBLK = 8
_BO = _N // _NBLK

_NEG = float("-inf")


def _body(field_ref, w_ref, b_ref, out_ref, scores_ref):
    i = pl.program_id(0)
    f = field_ref[...]
    wblk = w_ref[...]
    blk = lax.dot_general(
        f, wblk, (((1,), (1,)), ((), ())), preferred_element_type=jnp.float32
    )
    scores_ref[:, pl.ds(i * _BO, _BO)] = blk + b_ref[...].reshape(1, _BO)

    @pl.when(i == _NBLK - 1)
    def _select():
        s = scores_ref[...]
        col = lax.broadcasted_iota(jnp.int32, (_B, _N), 1)
        for k in range(_K):
            idx = jnp.argmax(s, axis=1)[:, None]
            hit = col == idx
            out_ref[:, k] = jnp.max(jnp.where(hit, f, _NEG), axis=1)
            if k < _K - 1:
                s = jnp.where(hit, _NEG, s)


def kernel(field, W, b, training):
    del training
    return pl.pallas_call(
        _body,
        grid=(_NBLK,),
        in_specs=[
            pl.BlockSpec((_B, _N), lambda i: (0, 0)),
            pl.BlockSpec((_BO, _N), lambda i: (i, 0)),
            pl.BlockSpec((_BO,), lambda i: (i,)),
        ],
        out_specs=pl.BlockSpec((_B, _K), lambda i: (0, 0)),
        out_shape=jax.ShapeDtypeStruct((_B, _K), jnp.float32),
        scratch_shapes=[pltpu.VMEM((_B, _N), jnp.float32)],
        compiler_params=pltpu.CompilerParams(
            dimension_semantics=("arbitrary",),
        ),
    )(field, W, b)
